# baseline (device time: 28384 ns/iter reference)
import jax
import jax.numpy as jnp
from jax import lax
from jax.experimental import pallas as pl
from jax.experimental.pallas import tpu as pltpu

N_DEV = 8
B_LOC = 2
SQ = 128
HQ = 32
H_BLK = 4
DH = 64
D_MODEL = 512
D_BLK = H_BLK * DH

_ARRIVAL_MASKS = [0, 1, 3, 4, 5, 2, 7, 6]


def kernel(x, Wq, K_ext, V_ext, Wo):
    my = lax.axis_index("i")

    wc = jnp.stack([Wq, Wo.T]).astype(jnp.bfloat16)

    K_loc = lax.dynamic_slice_in_dim(K_ext, my * B_LOC, B_LOC, axis=0)
    V_loc = lax.dynamic_slice_in_dim(V_ext, my * B_LOC, B_LOC, axis=0)
    order = my ^ jnp.array(_ARRIVAL_MASKS)
    K_loc = K_loc.astype(jnp.bfloat16).reshape(B_LOC, SQ, N_DEV, H_BLK * DH)
    V_loc = V_loc.astype(jnp.bfloat16).reshape(B_LOC, SQ, N_DEV, H_BLK * DH)
    K_loc = jnp.take(K_loc, order, axis=2).reshape(B_LOC, SQ, HQ * DH)
    V_loc = jnp.take(V_loc, order, axis=2).reshape(B_LOC, SQ, HQ * DH)

    def body(x_ref, wc_ref, k_ref, v_ref, out_ref,
             wcg, xs, xr, ys, yr, zs, zr):
        my_i = lax.axis_index("i")
        nx = my_i ^ 1
        ny = my_i ^ 3
        nz = my_i ^ 4

        barrier_sem = pltpu.get_barrier_semaphore()
        for nbr in (nx, ny, nz):
            pl.semaphore_signal(barrier_sem, inc=1, device_id=(nbr,),
                                device_id_type=pl.DeviceIdType.MESH)
        pl.semaphore_wait(barrier_sem, 3)

        def rc(src, dst, send_sem, recv_sem, dev):
            return pltpu.make_async_remote_copy(
                src_ref=src, dst_ref=dst, send_sem=send_sem,
                recv_sem=recv_sem, device_id=(dev,),
                device_id_type=pl.DeviceIdType.MESH)

        QR = [(qi // 2, slice(256 * (qi % 2), 256 * (qi % 2 + 1)))
              for qi in range(4)]

        def quarters(r0_dst, r1_src, r1_dst, ss, rr, dev):
            r0 = [rc(wc_ref.at[h, rs], wcg.at[r0_dst, h, rs],
                     ss.at[qi], rr.at[qi], dev)
                  for qi, (h, rs) in enumerate(QR)]
            r1 = [rc(wcg.at[r1_src, h, rs], wcg.at[r1_dst, h, rs],
                     ss.at[4 + qi], rr.at[4 + qi], dev)
                  for qi, (h, rs) in enumerate(QR)]
            return r0, r1

        r0x, r1x = quarters(0, 2, 3, xs, xr, nx)
        r0y, r1y = quarters(1, 0, 4, ys, yr, ny)
        r0z, r1z = quarters(2, 1, 5, zs, zr, nz)
        r2x = [rc(wcg.at[5, 0, rs], wcg.at[6, 0, rs],
                  xs.at[8 + j], xr.at[8 + j], nx)
               for j, (_, rs) in enumerate(QR[:2])]
        r2y = [rc(wcg.at[3, 1, rs], wcg.at[6, 1, rs],
                  ys.at[8 + j], yr.at[8 + j], ny)
               for j, (_, rs) in enumerate(QR[:2])]

        xb = x_ref[...].reshape(B_LOC * SQ, D_MODEL).astype(jnp.bfloat16)

        def contribution(wq_p, wot_p, blk):
            q = jnp.dot(xb, wq_p, preferred_element_type=jnp.float32)
            q = (q * 0.125).astype(jnp.bfloat16)
            rows = []
            for b in range(B_LOC):
                s_h = []
                for hh in range(H_BLK):
                    qh = q[b * SQ:(b + 1) * SQ, hh * DH:(hh + 1) * DH]
                    c0 = (blk * H_BLK + hh) * DH
                    s_h.append(lax.dot_general(
                        qh, k_ref[b, :, c0:c0 + DH], (((1,), (1,)), ((), ())),
                        preferred_element_type=jnp.float32))
                e = jnp.exp(jnp.concatenate(s_h, axis=1))
                denom = jnp.sum(e.reshape(SQ, H_BLK, SQ), axis=-1)
                scale = 1.0 / denom
                eb = e.astype(jnp.bfloat16)
                ctx_h = []
                for hh in range(H_BLK):
                    c0 = (blk * H_BLK + hh) * DH
                    ctx_raw = jnp.dot(
                        eb[:, hh * SQ:(hh + 1) * SQ], v_ref[b, :, c0:c0 + DH],
                        preferred_element_type=jnp.float32)
                    ctx_h.append(ctx_raw * scale[:, hh:hh + 1])
                rows.append(jnp.concatenate(ctx_h, axis=1))
            ctx = jnp.concatenate(rows, axis=0).astype(jnp.bfloat16)
            return lax.dot_general(
                ctx, wot_p, (((1,), (1,)), ((), ())),
                preferred_element_type=jnp.float32)

        slot_c = lambda s: contribution(wcg[s, 0], wcg[s, 1], 1 + s)

        for qi in range(4):
            r0x[qi].start()
            r0y[qi].start()
            r0z[qi].start()
        acc = contribution(wc_ref[0], wc_ref[1], 0)

        for qi in range(4):
            r0x[qi].wait_recv()
            r1y[qi].start()
            r0y[qi].wait_recv()
            r1z[qi].start()
            r0z[qi].wait_recv()
            r1x[qi].start()
        acc = acc + slot_c(0) + slot_c(1) + slot_c(2)

        r1z[0].wait_recv()
        r2x[0].start()
        r1z[1].wait_recv()
        r2x[1].start()
        r1x[2].wait_recv()
        r2y[0].start()
        r1x[3].wait_recv()
        r2y[1].start()
        for d in (r1x[0], r1x[1], r1y[0], r1y[1], r1y[2], r1y[3],
                  r1z[2], r1z[3]):
            d.wait_recv()
        acc = acc + slot_c(3) + slot_c(4) + slot_c(5)

        for d in r2x + r2y:
            d.wait_recv()
        acc = acc + slot_c(6)

        for d in r0x + r0y + r0z + r1x + r1y + r1z + r2x + r2y:
            d.wait_send()

        out_ref[...] = acc.reshape(B_LOC, SQ, D_MODEL)

    return pl.pallas_call(
        body,
        out_shape=jax.ShapeDtypeStruct((B_LOC, SQ, D_MODEL), jnp.float32),
        in_specs=[pl.BlockSpec(memory_space=pltpu.VMEM)] * 4,
        out_specs=pl.BlockSpec(memory_space=pltpu.VMEM),
        scratch_shapes=[
            pltpu.VMEM((7, 2, D_MODEL, D_BLK), jnp.bfloat16),
            pltpu.SemaphoreType.DMA((10,)),
            pltpu.SemaphoreType.DMA((10,)),
            pltpu.SemaphoreType.DMA((10,)),
            pltpu.SemaphoreType.DMA((10,)),
            pltpu.SemaphoreType.DMA((8,)),
            pltpu.SemaphoreType.DMA((8,)),
        ],
        compiler_params=pltpu.CompilerParams(collective_id=0),
    )(x, wc, K_loc, V_loc)
